# baseline (speedup 1.0000x reference)
import functools

import jax
import jax.numpy as jnp
from jax import lax
from jax.experimental import pallas as pl
from jax.experimental.pallas import tpu as pltpu
from jax.experimental.pallas import tpu_sc as plsc

VOCAB = 30522
HIDDEN = 1024
MAX_POS = 512
BATCH = 64
SEQ = 512
EPS = 1e-12

SLICES = (4, 20, 20, 20)
OFFSETS = tuple(sum(SLICES[:i]) for i in range(len(SLICES)))

NC = 2
NS = 16
NW = NC * NS
TOKENS = BATCH * SEQ
CH = 32


def _sc_gather_body(table_hbm, idx_hbm, out_hbm, idx_v, rows_v, gs0, gs1, os0, os1,
                    *, tok_w, nch, ng):
    wid = lax.axis_index("s") * NC + lax.axis_index("c")
    base = wid * tok_w
    pltpu.sync_copy(idx_hbm.at[pl.ds(wid * tok_w, tok_w)], idx_v)
    gsem = (gs0, gs1)
    osem = (os0, os1)

    def start_gather(c, p):
        pltpu.async_copy(table_hbm.at[idx_v.at[pl.ds(c * CH, CH)]], rows_v.at[p], gsem[p])

    def wait_gather(p):
        pltpu.make_async_copy(table_hbm.at[pl.ds(0, CH)], rows_v.at[p], gsem[p]).wait()

    def start_out(c, p):
        pltpu.async_copy(rows_v.at[p], out_hbm.at[pl.ds(base + c * CH, CH)], osem[p])

    def wait_out(p):
        pltpu.make_async_copy(out_hbm.at[pl.ds(0, CH)], rows_v.at[p], osem[p]).wait()

    start_gather(0, 0)
    start_gather(1, 1)

    def step(g, carry):
        for p in (0, 1):
            c = 2 * g + p
            wait_gather(p)
            start_out(c, p)

            @pl.when(g < ng - 1)
            def _():
                wait_out(p)
                start_gather(c + 2, p)

        return carry

    lax.fori_loop(0, ng, step, 0)
    wait_out(0)
    wait_out(1)


@functools.cache
def _make_sc_gather(n_tokens):
    tok_w = n_tokens // NW
    nch = tok_w // CH
    ng = nch // 2
    return pl.kernel(
        functools.partial(_sc_gather_body, tok_w=tok_w, nch=nch, ng=ng),
        mesh=plsc.VectorSubcoreMesh(core_axis_name="c", subcore_axis_name="s"),
        out_type=jax.ShapeDtypeStruct((n_tokens, HIDDEN), jnp.float32),
        scratch_types=[
            pltpu.VMEM((tok_w,), jnp.int32),
            pltpu.VMEM((2, CH, HIDDEN), jnp.float32),
            pltpu.SemaphoreType.DMA,
            pltpu.SemaphoreType.DMA,
            pltpu.SemaphoreType.DMA,
            pltpu.SemaphoreType.DMA,
        ],
    )


def _tc_compute(g_ref, tt_ref, pos_ref, type_ref, w_ref, b_ref, o_ref):
    x = g_ref[0]
    t0 = type_ref[0:1, :]
    t1 = type_ref[1:2, :]
    ttc = tt_ref[0]
    x = x + pos_ref[...] + t0 + ttc * (t1 - t0)
    u = jnp.mean(x, axis=1, keepdims=True)
    xc = x - u
    v = jnp.mean(xc * xc, axis=1, keepdims=True)
    y = xc * lax.rsqrt(v + EPS)
    y = y * w_ref[...] + b_ref[...]
    o_ref[0] = y.T


def _tc_body_first(g_ref, tt_ref, pos_ref, type_ref, w_ref, b_ref, o_ref):
    _tc_compute(g_ref, tt_ref, pos_ref, type_ref, w_ref, b_ref, o_ref)


def _tc_body_chained(prev_ref, g_ref, tt_ref, pos_ref, type_ref, w_ref, b_ref, o_ref):
    del prev_ref
    _tc_compute(g_ref, tt_ref, pos_ref, type_ref, w_ref, b_ref, o_ref)


def _tc_fuse_slice(k, prev_out, gathered, ttf, pos_emb, type_emb, w2, b2):
    bk = SLICES[k]
    b0 = OFFSETS[k]
    data_specs = [
        pl.BlockSpec((1, SEQ, HIDDEN), lambda b: (b, 0, 0)),
        pl.BlockSpec((1, SEQ, 1), lambda b: (b, 0, 0)),
        pl.BlockSpec((SEQ, HIDDEN), lambda b: (0, 0)),
        pl.BlockSpec((2, HIDDEN), lambda b: (0, 0)),
        pl.BlockSpec((1, HIDDEN), lambda b: (0, 0)),
        pl.BlockSpec((1, HIDDEN), lambda b: (0, 0)),
    ]
    out_spec = pl.BlockSpec((1, HIDDEN, SEQ), lambda b, _b0=b0: (_b0 + b, 0, 0))
    out_shape = jax.ShapeDtypeStruct((BATCH, HIDDEN, SEQ), jnp.float32)
    args = (gathered, ttf, pos_emb, type_emb, w2, b2)
    if k == 0:
        return pl.pallas_call(
            _tc_body_first,
            grid=(bk,),
            in_specs=data_specs,
            out_specs=out_spec,
            out_shape=out_shape,
        )(*args)
    return pl.pallas_call(
        _tc_body_chained,
        grid=(bk,),
        in_specs=[pl.BlockSpec(memory_space=pl.ANY)] + data_specs,
        out_specs=out_spec,
        out_shape=out_shape,
        input_output_aliases={0: 0},
    )(prev_out, *args)


def kernel(input_ids, token_type_ids, word_emb, pos_emb, type_emb, ln_weight, ln_bias):
    ids = input_ids.astype(jnp.int32)
    ttf = token_type_ids.astype(jnp.float32).reshape(BATCH, SEQ, 1)
    w2 = ln_weight.reshape(1, HIDDEN)
    b2 = ln_bias.reshape(1, HIDDEN)
    gathered = []
    for k, bk in enumerate(SLICES):
        b0 = OFFSETS[k]
        n_tok = bk * SEQ
        sc = _make_sc_gather(n_tok)
        gathered.append(sc(word_emb, ids[b0:b0 + bk].reshape(n_tok)))
    out = None
    for k, bk in enumerate(SLICES):
        b0 = OFFSETS[k]
        out = _tc_fuse_slice(
            k, out,
            gathered[k].reshape(bk, SEQ, HIDDEN),
            ttf[b0:b0 + bk],
            pos_emb, type_emb, w2, b2,
        )
    return out


# speedup vs baseline: 1.0022x; 1.0022x over previous
import functools

import jax
import jax.numpy as jnp
from jax import lax
from jax.experimental import pallas as pl
from jax.experimental.pallas import tpu as pltpu
from jax.experimental.pallas import tpu_sc as plsc

VOCAB = 30522
HIDDEN = 1024
MAX_POS = 512
BATCH = 64
SEQ = 512
EPS = 1e-12

SLICES = (4, 8, 16, 16, 20)
OFFSETS = tuple(sum(SLICES[:i]) for i in range(len(SLICES)))

NC = 2
NS = 16
NW = NC * NS
TOKENS = BATCH * SEQ
CH = 32


def _sc_gather_body(table_hbm, idx_hbm, out_hbm, idx_v, rows_v, gs0, gs1, os0, os1,
                    *, tok_w, nch, ng):
    wid = lax.axis_index("s") * NC + lax.axis_index("c")
    base = wid * tok_w
    pltpu.sync_copy(idx_hbm.at[pl.ds(wid * tok_w, tok_w)], idx_v)
    gsem = (gs0, gs1)
    osem = (os0, os1)

    def start_gather(c, p):
        pltpu.async_copy(table_hbm.at[idx_v.at[pl.ds(c * CH, CH)]], rows_v.at[p], gsem[p])

    def wait_gather(p):
        pltpu.make_async_copy(table_hbm.at[pl.ds(0, CH)], rows_v.at[p], gsem[p]).wait()

    def start_out(c, p):
        pltpu.async_copy(rows_v.at[p], out_hbm.at[pl.ds(base + c * CH, CH)], osem[p])

    def wait_out(p):
        pltpu.make_async_copy(out_hbm.at[pl.ds(0, CH)], rows_v.at[p], osem[p]).wait()

    start_gather(0, 0)
    start_gather(1, 1)

    def step(g, carry):
        for p in (0, 1):
            c = 2 * g + p
            wait_gather(p)
            start_out(c, p)

            @pl.when(g < ng - 1)
            def _():
                wait_out(p)
                start_gather(c + 2, p)

        return carry

    lax.fori_loop(0, ng, step, 0)
    wait_out(0)
    wait_out(1)


@functools.cache
def _make_sc_gather(n_tokens):
    tok_w = n_tokens // NW
    nch = tok_w // CH
    ng = nch // 2
    return pl.kernel(
        functools.partial(_sc_gather_body, tok_w=tok_w, nch=nch, ng=ng),
        mesh=plsc.VectorSubcoreMesh(core_axis_name="c", subcore_axis_name="s"),
        out_type=jax.ShapeDtypeStruct((n_tokens, HIDDEN), jnp.float32),
        scratch_types=[
            pltpu.VMEM((tok_w,), jnp.int32),
            pltpu.VMEM((2, CH, HIDDEN), jnp.float32),
            pltpu.SemaphoreType.DMA,
            pltpu.SemaphoreType.DMA,
            pltpu.SemaphoreType.DMA,
            pltpu.SemaphoreType.DMA,
        ],
    )


def _tc_compute(g_ref, tt_ref, pos_ref, type_ref, w_ref, b_ref, o_ref):
    x = g_ref[0]
    t0 = type_ref[0:1, :]
    t1 = type_ref[1:2, :]
    ttc = tt_ref[0]
    x = x + pos_ref[...] + t0 + ttc * (t1 - t0)
    u = jnp.mean(x, axis=1, keepdims=True)
    xc = x - u
    v = jnp.mean(xc * xc, axis=1, keepdims=True)
    y = xc * lax.rsqrt(v + EPS)
    y = y * w_ref[...] + b_ref[...]
    o_ref[0] = y.T


def _tc_body_first(g_ref, tt_ref, pos_ref, type_ref, w_ref, b_ref, o_ref):
    _tc_compute(g_ref, tt_ref, pos_ref, type_ref, w_ref, b_ref, o_ref)


def _tc_body_chained(prev_ref, g_ref, tt_ref, pos_ref, type_ref, w_ref, b_ref, o_ref):
    del prev_ref
    _tc_compute(g_ref, tt_ref, pos_ref, type_ref, w_ref, b_ref, o_ref)


def _tc_fuse_slice(k, prev_out, gathered, ttf, pos_emb, type_emb, w2, b2):
    bk = SLICES[k]
    b0 = OFFSETS[k]
    data_specs = [
        pl.BlockSpec((1, SEQ, HIDDEN), lambda b: (b, 0, 0)),
        pl.BlockSpec((1, SEQ, 1), lambda b: (b, 0, 0)),
        pl.BlockSpec((SEQ, HIDDEN), lambda b: (0, 0)),
        pl.BlockSpec((2, HIDDEN), lambda b: (0, 0)),
        pl.BlockSpec((1, HIDDEN), lambda b: (0, 0)),
        pl.BlockSpec((1, HIDDEN), lambda b: (0, 0)),
    ]
    out_spec = pl.BlockSpec((1, HIDDEN, SEQ), lambda b, _b0=b0: (_b0 + b, 0, 0))
    out_shape = jax.ShapeDtypeStruct((BATCH, HIDDEN, SEQ), jnp.float32)
    args = (gathered, ttf, pos_emb, type_emb, w2, b2)
    if k == 0:
        return pl.pallas_call(
            _tc_body_first,
            grid=(bk,),
            in_specs=data_specs,
            out_specs=out_spec,
            out_shape=out_shape,
        )(*args)
    return pl.pallas_call(
        _tc_body_chained,
        grid=(bk,),
        in_specs=[pl.BlockSpec(memory_space=pl.ANY)] + data_specs,
        out_specs=out_spec,
        out_shape=out_shape,
        input_output_aliases={0: 0},
    )(prev_out, *args)


def kernel(input_ids, token_type_ids, word_emb, pos_emb, type_emb, ln_weight, ln_bias):
    ids = input_ids.astype(jnp.int32)
    ttf = token_type_ids.astype(jnp.float32).reshape(BATCH, SEQ, 1)
    w2 = ln_weight.reshape(1, HIDDEN)
    b2 = ln_bias.reshape(1, HIDDEN)
    gathered = []
    for k, bk in enumerate(SLICES):
        b0 = OFFSETS[k]
        n_tok = bk * SEQ
        sc = _make_sc_gather(n_tok)
        gathered.append(sc(word_emb, ids[b0:b0 + bk].reshape(n_tok)))
    out = None
    for k, bk in enumerate(SLICES):
        b0 = OFFSETS[k]
        out = _tc_fuse_slice(
            k, out,
            gathered[k].reshape(bk, SEQ, HIDDEN),
            ttf[b0:b0 + bk],
            pos_emb, type_emb, w2, b2,
        )
    return out


# speedup vs baseline: 1.0635x; 1.0612x over previous
import functools

import jax
import jax.numpy as jnp
from jax import lax
from jax.experimental import pallas as pl
from jax.experimental.pallas import tpu as pltpu
from jax.experimental.pallas import tpu_sc as plsc

VOCAB = 30522
HIDDEN = 1024
MAX_POS = 512
BATCH = 64
SEQ = 512
EPS = 1e-12

SLICES = (16, 16, 16, 16)
OFFSETS = tuple(sum(SLICES[:i]) for i in range(len(SLICES)))

NC = 2
NS = 16
NW = NC * NS
TOKENS = BATCH * SEQ
CH = 32


def _sc_gather_body(table_hbm, idx_hbm, out_hbm, idx_v, rows_v, gs0, gs1, os0, os1,
                    *, tok_w, nch, ng):
    wid = lax.axis_index("s") * NC + lax.axis_index("c")
    base = wid * tok_w
    pltpu.sync_copy(idx_hbm.at[pl.ds(wid * tok_w, tok_w)], idx_v)
    gsem = (gs0, gs1)
    osem = (os0, os1)

    def start_gather(c, p):
        pltpu.async_copy(table_hbm.at[idx_v.at[pl.ds(c * CH, CH)]], rows_v.at[p], gsem[p])

    def wait_gather(p):
        pltpu.make_async_copy(table_hbm.at[pl.ds(0, CH)], rows_v.at[p], gsem[p]).wait()

    def start_out(c, p):
        pltpu.async_copy(rows_v.at[p], out_hbm.at[pl.ds(base + c * CH, CH)], osem[p])

    def wait_out(p):
        pltpu.make_async_copy(out_hbm.at[pl.ds(0, CH)], rows_v.at[p], osem[p]).wait()

    start_gather(0, 0)
    start_gather(1, 1)

    def step(g, carry):
        for p in (0, 1):
            c = 2 * g + p
            wait_gather(p)
            start_out(c, p)

            @pl.when(g < ng - 1)
            def _():
                wait_out(p)
                start_gather(c + 2, p)

        return carry

    lax.fori_loop(0, ng, step, 0)
    wait_out(0)
    wait_out(1)


@functools.cache
def _make_sc_gather(n_tokens):
    tok_w = n_tokens // NW
    nch = tok_w // CH
    ng = nch // 2
    return pl.kernel(
        functools.partial(_sc_gather_body, tok_w=tok_w, nch=nch, ng=ng),
        mesh=plsc.VectorSubcoreMesh(core_axis_name="c", subcore_axis_name="s"),
        out_type=jax.ShapeDtypeStruct((n_tokens, HIDDEN), jnp.float32),
        scratch_types=[
            pltpu.VMEM((tok_w,), jnp.int32),
            pltpu.VMEM((2, CH, HIDDEN), jnp.float32),
            pltpu.SemaphoreType.DMA,
            pltpu.SemaphoreType.DMA,
            pltpu.SemaphoreType.DMA,
            pltpu.SemaphoreType.DMA,
        ],
    )


def _tc_compute(g_ref, tt_ref, pos_ref, type_ref, w_ref, b_ref, o_ref):
    x = g_ref[...]
    t0 = type_ref[0:1, :]
    t1 = type_ref[1:2, :]
    ttc = tt_ref[...]
    x = x + pos_ref[...] + t0 + ttc * (t1 - t0)
    u = jnp.mean(x, axis=2, keepdims=True)
    xc = x - u
    v = jnp.mean(xc * xc, axis=2, keepdims=True)
    y = xc * lax.rsqrt(v + EPS)
    y = y * w_ref[...] + b_ref[...]
    o_ref[0] = y[0].T
    o_ref[1] = y[1].T


def _tc_body_first(g_ref, tt_ref, pos_ref, type_ref, w_ref, b_ref, o_ref):
    _tc_compute(g_ref, tt_ref, pos_ref, type_ref, w_ref, b_ref, o_ref)


def _tc_body_chained(prev_ref, g_ref, tt_ref, pos_ref, type_ref, w_ref, b_ref, o_ref):
    del prev_ref
    _tc_compute(g_ref, tt_ref, pos_ref, type_ref, w_ref, b_ref, o_ref)


def _tc_fuse_slice(k, prev_out, gathered, ttf, pos_emb, type_emb, w2, b2):
    bk = SLICES[k]
    b0 = OFFSETS[k]
    data_specs = [
        pl.BlockSpec((2, SEQ, HIDDEN), lambda b: (b, 0, 0)),
        pl.BlockSpec((2, SEQ, 1), lambda b: (b, 0, 0)),
        pl.BlockSpec((SEQ, HIDDEN), lambda b: (0, 0)),
        pl.BlockSpec((2, HIDDEN), lambda b: (0, 0)),
        pl.BlockSpec((1, HIDDEN), lambda b: (0, 0)),
        pl.BlockSpec((1, HIDDEN), lambda b: (0, 0)),
    ]
    out_spec = pl.BlockSpec((2, HIDDEN, SEQ), lambda b, _b0=b0: (_b0 // 2 + b, 0, 0))
    out_shape = jax.ShapeDtypeStruct((BATCH, HIDDEN, SEQ), jnp.float32)
    args = (gathered, ttf, pos_emb, type_emb, w2, b2)
    if k == 0:
        return pl.pallas_call(
            _tc_body_first,
            grid=(bk // 2,),
            in_specs=data_specs,
            out_specs=out_spec,
            out_shape=out_shape,
        )(*args)
    return pl.pallas_call(
        _tc_body_chained,
        grid=(bk // 2,),
        in_specs=[pl.BlockSpec(memory_space=pl.ANY)] + data_specs,
        out_specs=out_spec,
        out_shape=out_shape,
        input_output_aliases={0: 0},
    )(prev_out, *args)


def kernel(input_ids, token_type_ids, word_emb, pos_emb, type_emb, ln_weight, ln_bias):
    ids = input_ids.astype(jnp.int32)
    ttf = token_type_ids.astype(jnp.float32).reshape(BATCH, SEQ, 1)
    w2 = ln_weight.reshape(1, HIDDEN)
    b2 = ln_bias.reshape(1, HIDDEN)
    gathered = []
    for k, bk in enumerate(SLICES):
        b0 = OFFSETS[k]
        n_tok = bk * SEQ
        sc = _make_sc_gather(n_tok)
        gathered.append(sc(word_emb, ids[b0:b0 + bk].reshape(n_tok)))
    out = None
    for k, bk in enumerate(SLICES):
        b0 = OFFSETS[k]
        out = _tc_fuse_slice(
            k, out,
            gathered[k].reshape(bk, SEQ, HIDDEN),
            ttf[b0:b0 + bk],
            pos_emb, type_emb, w2, b2,
        )
    return out


# speedup vs baseline: 1.0858x; 1.0210x over previous
import functools

import jax
import jax.numpy as jnp
from jax import lax
from jax.experimental import pallas as pl
from jax.experimental.pallas import tpu as pltpu
from jax.experimental.pallas import tpu_sc as plsc

VOCAB = 30522
HIDDEN = 1024
MAX_POS = 512
BATCH = 64
SEQ = 512
EPS = 1e-12

SLICES = (16, 16, 16, 16)
OFFSETS = tuple(sum(SLICES[:i]) for i in range(len(SLICES)))

NC = 2
NS = 16
NW = NC * NS
TOKENS = BATCH * SEQ
CH = 32


def _sc_gather_body(table_hbm, idx_hbm, out_hbm, idx_v, rows_v, gs0, gs1, os0, os1,
                    *, tok_w, nch, ng):
    wid = lax.axis_index("s") * NC + lax.axis_index("c")
    base = wid * tok_w
    pltpu.sync_copy(idx_hbm.at[pl.ds(wid * tok_w, tok_w)], idx_v)
    gsem = (gs0, gs1)
    osem = (os0, os1)

    def start_gather(c, p):
        pltpu.async_copy(table_hbm.at[idx_v.at[pl.ds(c * CH, CH)]], rows_v.at[p], gsem[p])

    def wait_gather(p):
        pltpu.make_async_copy(table_hbm.at[pl.ds(0, CH)], rows_v.at[p], gsem[p]).wait()

    def start_out(c, p):
        pltpu.async_copy(rows_v.at[p], out_hbm.at[pl.ds(base + c * CH, CH)], osem[p])

    def wait_out(p):
        pltpu.make_async_copy(out_hbm.at[pl.ds(0, CH)], rows_v.at[p], osem[p]).wait()

    start_gather(0, 0)
    start_gather(1, 1)

    def step(g, carry):
        for p in (0, 1):
            c = 2 * g + p
            wait_gather(p)
            start_out(c, p)

            @pl.when(g < ng - 1)
            def _():
                wait_out(p)
                start_gather(c + 2, p)

        return carry

    lax.fori_loop(0, ng, step, 0)
    wait_out(0)
    wait_out(1)


@functools.cache
def _make_sc_gather(n_tokens):
    tok_w = n_tokens // NW
    nch = tok_w // CH
    ng = nch // 2
    return pl.kernel(
        functools.partial(_sc_gather_body, tok_w=tok_w, nch=nch, ng=ng),
        mesh=plsc.VectorSubcoreMesh(core_axis_name="c", subcore_axis_name="s"),
        out_type=jax.ShapeDtypeStruct((n_tokens, HIDDEN), jnp.float32),
        scratch_types=[
            pltpu.VMEM((tok_w,), jnp.int32),
            pltpu.VMEM((2, CH, HIDDEN), jnp.float32),
            pltpu.SemaphoreType.DMA,
            pltpu.SemaphoreType.DMA,
            pltpu.SemaphoreType.DMA,
            pltpu.SemaphoreType.DMA,
        ],
    )


NB = 4

def _tc_compute(g_ref, tt_ref, pos_ref, type_ref, w_ref, b_ref, o_ref):
    x = g_ref[...]
    t0 = type_ref[0:1, :]
    t1 = type_ref[1:2, :]
    ttc = tt_ref[...]
    x = x + pos_ref[...] + t0 + ttc * (t1 - t0)
    u = jnp.mean(x, axis=2, keepdims=True)
    xc = x - u
    v = jnp.mean(xc * xc, axis=2, keepdims=True)
    y = xc * lax.rsqrt(v + EPS)
    y = y * w_ref[...] + b_ref[...]
    for i in range(NB):
        o_ref[i] = y[i].T


def _tc_body_first(g_ref, tt_ref, pos_ref, type_ref, w_ref, b_ref, o_ref):
    _tc_compute(g_ref, tt_ref, pos_ref, type_ref, w_ref, b_ref, o_ref)


def _tc_body_chained(prev_ref, g_ref, tt_ref, pos_ref, type_ref, w_ref, b_ref, o_ref):
    del prev_ref
    _tc_compute(g_ref, tt_ref, pos_ref, type_ref, w_ref, b_ref, o_ref)


def _tc_fuse_slice(k, prev_out, gathered, ttf, pos_emb, type_emb, w2, b2):
    bk = SLICES[k]
    b0 = OFFSETS[k]
    data_specs = [
        pl.BlockSpec((NB, SEQ, HIDDEN), lambda b: (b, 0, 0)),
        pl.BlockSpec((NB, SEQ, 1), lambda b: (b, 0, 0)),
        pl.BlockSpec((SEQ, HIDDEN), lambda b: (0, 0)),
        pl.BlockSpec((2, HIDDEN), lambda b: (0, 0)),
        pl.BlockSpec((1, HIDDEN), lambda b: (0, 0)),
        pl.BlockSpec((1, HIDDEN), lambda b: (0, 0)),
    ]
    out_spec = pl.BlockSpec((NB, HIDDEN, SEQ), lambda b, _b0=b0: (_b0 // NB + b, 0, 0))
    out_shape = jax.ShapeDtypeStruct((BATCH, HIDDEN, SEQ), jnp.float32)
    args = (gathered, ttf, pos_emb, type_emb, w2, b2)
    if k == 0:
        return pl.pallas_call(
            _tc_body_first,
            grid=(bk // NB,),
            in_specs=data_specs,
            out_specs=out_spec,
            out_shape=out_shape,
        )(*args)
    return pl.pallas_call(
        _tc_body_chained,
        grid=(bk // NB,),
        in_specs=[pl.BlockSpec(memory_space=pl.ANY)] + data_specs,
        out_specs=out_spec,
        out_shape=out_shape,
        input_output_aliases={0: 0},
    )(prev_out, *args)


def kernel(input_ids, token_type_ids, word_emb, pos_emb, type_emb, ln_weight, ln_bias):
    ids = input_ids.astype(jnp.int32)
    ttf = token_type_ids.astype(jnp.float32).reshape(BATCH, SEQ, 1)
    w2 = ln_weight.reshape(1, HIDDEN)
    b2 = ln_bias.reshape(1, HIDDEN)
    gathered = []
    for k, bk in enumerate(SLICES):
        b0 = OFFSETS[k]
        n_tok = bk * SEQ
        sc = _make_sc_gather(n_tok)
        gathered.append(sc(word_emb, ids[b0:b0 + bk].reshape(n_tok)))
    out = None
    for k, bk in enumerate(SLICES):
        b0 = OFFSETS[k]
        out = _tc_fuse_slice(
            k, out,
            gathered[k].reshape(bk, SEQ, HIDDEN),
            ttf[b0:b0 + bk],
            pos_emb, type_emb, w2, b2,
        )
    return out
